# baseline (speedup 1.0000x reference)
import functools

import jax
import jax.numpy as jnp
from jax import lax
from jax.experimental import pallas as pl
from jax.experimental.pallas import tpu as pltpu
from jax.experimental.pallas import tpu_sc as plsc


def _transpose_body(keys_ref, out_ref):
    out_ref[...] = keys_ref[...].T


def _sc_body(keysT, indices, labels, buffer, midx, mlab, ptr,
             outb, outi, outl, ptr_v):
    f, K = buffer.shape
    B = keysT.shape[1]
    nc = 2
    ns = 16
    nw = nc * ns
    rows = f // nw
    chunk = K // nw

    wid = lax.axis_index("s") * nc + lax.axis_index("c")
    r0 = wid * rows
    e0 = wid * chunk

    pltpu.sync_copy(ptr, ptr_v.at[pl.ds(0, 1)])
    p = ptr_v[...][0]
    p = jnp.clip(p, 0, K - B)
    p = pl.multiple_of(p, B)

    pltpu.sync_copy(buffer.at[pl.ds(r0, rows), :],
                    outb.at[pl.ds(r0, rows), :])
    pltpu.sync_copy(keysT.at[pl.ds(r0, rows), :],
                    outb.at[pl.ds(r0, rows), pl.ds(p, B)])

    in_slab = jnp.logical_and(e0 >= p, e0 < p + B)

    @pl.when(in_slab)
    def _():
        pltpu.sync_copy(indices.at[pl.ds(e0 - p, chunk)],
                        outi.at[pl.ds(e0, chunk)])
        pltpu.sync_copy(labels.at[pl.ds(e0 - p, chunk)],
                        outl.at[pl.ds(e0, chunk)])

    @pl.when(jnp.logical_not(in_slab))
    def _():
        pltpu.sync_copy(midx.at[pl.ds(e0, chunk)],
                        outi.at[pl.ds(e0, chunk)])
        pltpu.sync_copy(mlab.at[pl.ds(e0, chunk)],
                        outl.at[pl.ds(e0, chunk)])


def kernel(keys, indices, labels, buffer, mem_indices, mem_labels, ptr,
           num_updates):
    f, K = buffer.shape
    B = keys.shape[0]

    keysT = pl.pallas_call(
        _transpose_body,
        out_shape=jax.ShapeDtypeStruct((f, B), keys.dtype),
    )(keys)

    mesh = plsc.VectorSubcoreMesh(core_axis_name="c", subcore_axis_name="s")
    sck = pl.kernel(
        _sc_body,
        out_type=[
            jax.ShapeDtypeStruct((f, K), buffer.dtype),
            jax.ShapeDtypeStruct((K,), mem_indices.dtype),
            jax.ShapeDtypeStruct((K,), mem_labels.dtype),
        ],
        mesh=mesh,
        scratch_types=[pltpu.VMEM((16,), jnp.int32)],
    )
    new_buffer, new_indices, new_labels = sck(
        keysT, indices, labels, buffer, mem_indices, mem_labels, ptr)

    p = ptr[0]
    is_reliable = (p + B) >= K
    new_ptr = jnp.reshape(((p + B) % K).astype(ptr.dtype), (1,))
    new_num_updates = num_updates + 1
    return (new_buffer, new_indices, new_labels, new_ptr, new_num_updates,
            is_reliable)


# speedup vs baseline: 22.5219x; 22.5219x over previous
import jax
import jax.numpy as jnp
from jax import lax
from jax.experimental import pallas as pl
from jax.experimental.pallas import tpu as pltpu
from jax.experimental.pallas import tpu_sc as plsc

_NC = 2
_NS = 16
_NW = _NC * _NS
_CC = 256


def _transpose_body(keys_ref, out_ref):
    out_ref[...] = keys_ref[...].T


def _sc_body(keysT, indices, labels, buffer, midx, mlab, ptr,
             outb, outi, outl,
             ptr_v, ring0, ring1, idx_v, lab_v,
             gsem0, gsem1, ssem0, ssem1, msem):
    f, K = buffer.shape
    B = keysT.shape[1]
    span = K // _NW
    nchunks = span // _CC
    ring = (ring0, ring1)
    gsem = (gsem0, gsem1)
    ssem = (ssem0, ssem1)

    wid = lax.axis_index("s") * _NC + lax.axis_index("c")
    c0 = wid * span

    pltpu.sync_copy(ptr, ptr_v.at[pl.ds(0, 1)])
    p = ptr_v[...][0]
    p = jnp.clip(p, 0, K - B)
    p = pl.multiple_of(p, 128)

    in_slab = jnp.logical_and(c0 >= p, c0 < p + B)

    def copy_span(src, src_c0):
        src_c0 = pl.multiple_of(src_c0, 128)

        def gather(i, b):
            return pltpu.async_copy(
                src.at[:, pl.ds(src_c0 + i * _CC, _CC)], ring[b], gsem[b])

        def scatter(i, b):
            return pltpu.async_copy(
                ring[b], outb.at[:, pl.ds(c0 + i * _CC, _CC)], ssem[b])

        h_in = [None] * nchunks
        h_out = [None] * nchunks
        h_in[0] = gather(0, 0)
        for i in range(nchunks):
            b = i % 2
            if i + 1 < nchunks:
                nb = (i + 1) % 2
                if i >= 1:
                    h_out[i - 1].wait()
                h_in[i + 1] = gather(i + 1, nb)
            h_in[i].wait()
            h_out[i] = scatter(i, b)
        if nchunks >= 2:
            h_out[nchunks - 2].wait()
        h_out[nchunks - 1].wait()

    @pl.when(in_slab)
    def _():
        copy_span(keysT, c0 - p)
        pltpu.sync_copy(indices.at[pl.ds(c0 - p, span)], idx_v)
        pltpu.sync_copy(labels.at[pl.ds(c0 - p, span)], lab_v)

    @pl.when(jnp.logical_not(in_slab))
    def _():
        copy_span(buffer, c0)
        pltpu.sync_copy(midx.at[pl.ds(c0, span)], idx_v)
        pltpu.sync_copy(mlab.at[pl.ds(c0, span)], lab_v)

    hi = pltpu.async_copy(idx_v, outi.at[pl.ds(c0, span)], msem)
    hl = pltpu.async_copy(lab_v, outl.at[pl.ds(c0, span)], msem)
    hi.wait()
    hl.wait()


def kernel(keys, indices, labels, buffer, mem_indices, mem_labels, ptr,
           num_updates):
    f, K = buffer.shape
    B = keys.shape[0]

    keysT = pl.pallas_call(
        _transpose_body,
        out_shape=jax.ShapeDtypeStruct((f, B), keys.dtype),
    )(keys)

    mesh = plsc.VectorSubcoreMesh(core_axis_name="c", subcore_axis_name="s")
    sck = pl.kernel(
        _sc_body,
        out_type=[
            jax.ShapeDtypeStruct((f, K), buffer.dtype),
            jax.ShapeDtypeStruct((K,), mem_indices.dtype),
            jax.ShapeDtypeStruct((K,), mem_labels.dtype),
        ],
        mesh=mesh,
        scratch_types=[
            pltpu.VMEM((16,), jnp.int32),
            pltpu.VMEM((f, _CC), jnp.float32),
            pltpu.VMEM((f, _CC), jnp.float32),
            pltpu.VMEM((K // _NW,), jnp.int32),
            pltpu.VMEM((K // _NW,), jnp.int32),
            pltpu.SemaphoreType.DMA,
            pltpu.SemaphoreType.DMA,
            pltpu.SemaphoreType.DMA,
            pltpu.SemaphoreType.DMA,
            pltpu.SemaphoreType.DMA,
        ],
    )
    new_buffer, new_indices, new_labels = sck(
        keysT, indices, labels, buffer, mem_indices, mem_labels, ptr)

    p = ptr[0]
    is_reliable = (p + B) >= K
    new_ptr = jnp.reshape(((p + B) % K).astype(ptr.dtype), (1,))
    new_num_updates = num_updates + 1
    return (new_buffer, new_indices, new_labels, new_ptr, new_num_updates,
            is_reliable)
